# Initial kernel scaffold; baseline (speedup 1.0000x reference)
import jax
import jax.numpy as jnp
from jax.experimental import pallas as pl


def kernel(x):
    raise NotImplementedError("write your pallas kernel here")


# speedup vs baseline: 4.6106x; 4.6106x over previous
import jax
import jax.numpy as jnp
from jax.experimental import pallas as pl

_ROWS = 8192
_COLS = 4096
_BLOCK_ROWS = 256
_CHUNK = 128


def _cumsum_kernel(x_ref, o_ref):
    r = jax.lax.broadcasted_iota(jnp.int32, (_CHUNK, _CHUNK), 0)
    c = jax.lax.broadcasted_iota(jnp.int32, (_CHUNK, _CHUNK), 1)
    tri = (r <= c).astype(jnp.float32)
    carry = jnp.zeros((_BLOCK_ROWS, 1), jnp.float32)
    for i in range(_COLS // _CHUNK):
        xc = x_ref[:, i * _CHUNK:(i + 1) * _CHUNK]
        y = jax.lax.dot(xc, tri, precision=jax.lax.Precision.HIGHEST)
        y = y + carry
        o_ref[:, i * _CHUNK:(i + 1) * _CHUNK] = y
        carry = y[:, _CHUNK - 1:_CHUNK]


def kernel(x):
    return pl.pallas_call(
        _cumsum_kernel,
        grid=(_ROWS // _BLOCK_ROWS,),
        in_specs=[pl.BlockSpec((_BLOCK_ROWS, _COLS), lambda i: (i, 0))],
        out_specs=pl.BlockSpec((_BLOCK_ROWS, _COLS), lambda i: (i, 0)),
        out_shape=jax.ShapeDtypeStruct((_ROWS, _COLS), x.dtype),
    )(x)


# speedup vs baseline: 5.4916x; 1.1911x over previous
import jax
import jax.numpy as jnp
from jax.experimental import pallas as pl
from jax.experimental.pallas import tpu as pltpu

_ROWS = 8192
_COLS = 4096
_BLOCK_ROWS = 256
_CHUNK = 128


def _cumsum_kernel(x_ref, o_ref):
    r = jax.lax.broadcasted_iota(jnp.int32, (_CHUNK, _CHUNK), 0)
    c = jax.lax.broadcasted_iota(jnp.int32, (_CHUNK, _CHUNK), 1)
    tri = (r <= c).astype(jnp.float32)
    carry = jnp.zeros((_BLOCK_ROWS, 1), jnp.float32)
    for i in range(_COLS // _CHUNK):
        xc = x_ref[:, i * _CHUNK:(i + 1) * _CHUNK]
        y = jax.lax.dot(xc, tri, precision=jax.lax.Precision.DEFAULT)
        y = y + carry
        o_ref[:, i * _CHUNK:(i + 1) * _CHUNK] = y
        carry = y[:, _CHUNK - 1:_CHUNK]


def kernel(x):
    return pl.pallas_call(
        _cumsum_kernel,
        grid=(_ROWS // _BLOCK_ROWS,),
        in_specs=[pl.BlockSpec((_BLOCK_ROWS, _COLS), lambda i: (i, 0))],
        out_specs=pl.BlockSpec((_BLOCK_ROWS, _COLS), lambda i: (i, 0)),
        out_shape=jax.ShapeDtypeStruct((_ROWS, _COLS), x.dtype),
        compiler_params=pltpu.CompilerParams(
            dimension_semantics=("parallel",),
        ),
    )(x)


# speedup vs baseline: 6.2663x; 1.1411x over previous
import jax
import jax.numpy as jnp
from jax.experimental import pallas as pl
from jax.experimental.pallas import tpu as pltpu

_ROWS = 8192
_COLS = 4096
_BLOCK_ROWS = 256
_CHUNK = 128


def _cumsum_kernel(x_ref, o_ref):
    r = jax.lax.broadcasted_iota(jnp.int32, (_CHUNK, _CHUNK), 0)
    c = jax.lax.broadcasted_iota(jnp.int32, (_CHUNK, _CHUNK), 1)
    tri = (r <= c).astype(jnp.float32)
    carry = jnp.zeros((_BLOCK_ROWS, 1), jnp.float32)
    for i in range(_COLS // _CHUNK):
        xc = x_ref[:, i * _CHUNK:(i + 1) * _CHUNK]
        m = jax.lax.dot(xc, tri, precision=jax.lax.Precision.DEFAULT)
        o_ref[:, i * _CHUNK:(i + 1) * _CHUNK] = m + carry
        carry = carry + m[:, _CHUNK - 1:_CHUNK]


def kernel(x):
    return pl.pallas_call(
        _cumsum_kernel,
        grid=(_ROWS // _BLOCK_ROWS,),
        in_specs=[pl.BlockSpec((_BLOCK_ROWS, _COLS), lambda i: (i, 0))],
        out_specs=pl.BlockSpec((_BLOCK_ROWS, _COLS), lambda i: (i, 0)),
        out_shape=jax.ShapeDtypeStruct((_ROWS, _COLS), x.dtype),
        compiler_params=pltpu.CompilerParams(
            dimension_semantics=("parallel",),
        ),
    )(x)


# speedup vs baseline: 6.5506x; 1.0454x over previous
import jax
import jax.numpy as jnp
from jax.experimental import pallas as pl
from jax.experimental.pallas import tpu as pltpu

_ROWS = 8192
_COLS = 4096
_BLOCK_ROWS = 512
_CHUNK = 128


def _cumsum_kernel(x_ref, o_ref):
    r = jax.lax.broadcasted_iota(jnp.int32, (_CHUNK, _CHUNK), 0)
    c = jax.lax.broadcasted_iota(jnp.int32, (_CHUNK, _CHUNK), 1)
    tri = (r <= c).astype(jnp.float32)
    carry = jnp.zeros((_BLOCK_ROWS, 1), jnp.float32)
    for i in range(_COLS // _CHUNK):
        xc = x_ref[:, i * _CHUNK:(i + 1) * _CHUNK]
        m = jax.lax.dot(xc, tri, precision=jax.lax.Precision.DEFAULT)
        o_ref[:, i * _CHUNK:(i + 1) * _CHUNK] = m + carry
        carry = carry + m[:, _CHUNK - 1:_CHUNK]


def kernel(x):
    return pl.pallas_call(
        _cumsum_kernel,
        grid=(_ROWS // _BLOCK_ROWS,),
        in_specs=[pl.BlockSpec((_BLOCK_ROWS, _COLS), lambda i: (i, 0))],
        out_specs=pl.BlockSpec((_BLOCK_ROWS, _COLS), lambda i: (i, 0)),
        out_shape=jax.ShapeDtypeStruct((_ROWS, _COLS), x.dtype),
        compiler_params=pltpu.CompilerParams(
            dimension_semantics=("parallel",),
        ),
    )(x)
